# Initial kernel scaffold; baseline (speedup 1.0000x reference)
import jax
import jax.numpy as jnp
from jax.experimental import pallas as pl


def kernel(x, edge_index):
    raise NotImplementedError("write your pallas kernel here")


# speedup vs baseline: 1.1279x; 1.1279x over previous
import functools

import jax
import jax.numpy as jnp
from jax import lax
from jax.experimental import pallas as pl
from jax.experimental.pallas import tpu as pltpu
from jax.experimental.pallas import tpu_sc as plsc

_E = 128


@functools.lru_cache(maxsize=None)
def _build(n_edges, n_nodes, d):
    info = plsc.get_sparse_core_info()
    ncores, nsub = info.num_cores, info.num_subcores
    nw = ncores * nsub
    n_chunks = n_edges // _E
    k_max = (n_chunks + nw - 1) // nw
    mesh = plsc.VectorSubcoreMesh(core_axis_name="c", subcore_axis_name="s")

    @functools.partial(
        pl.kernel,
        out_type=jax.ShapeDtypeStruct((n_edges,), jnp.float32),
        mesh=mesh,
        compiler_params=pltpu.CompilerParams(needs_layout_passes=False),
        scratch_types=[
            pltpu.VMEM((_E,), jnp.int32),
            pltpu.VMEM((_E,), jnp.int32),
            pltpu.VMEM((_E, d), jnp.float32),
            pltpu.VMEM((_E, d), jnp.float32),
            pltpu.VMEM((_E,), jnp.float32),
            pltpu.SemaphoreType.DMA,
        ],
    )
    def edge_dot(x_hbm, src_hbm, dst_hbm, out_hbm,
                 sidx, didx, srows, drows, outv, sem):
        wid = lax.axis_index("s") * ncores + lax.axis_index("c")
        lanes = lax.iota(jnp.int32, 16)

        def chunk_body(k, carry):
            ci = wid + k * nw

            @pl.when(ci < n_chunks)
            def _():
                base = ci * _E
                pltpu.sync_copy(src_hbm.at[pl.ds(base, _E)], sidx)
                pltpu.sync_copy(dst_hbm.at[pl.ds(base, _E)], didx)
                c1 = pltpu.async_copy(x_hbm.at[sidx], srows, sem)
                c2 = pltpu.async_copy(x_hbm.at[didx], drows, sem)
                c1.wait()
                c2.wait()

                def group_body(g, carry2):
                    row = g * 16 + lanes
                    acc = jnp.zeros((16,), jnp.float32)
                    for dd in range(d):
                        col = jnp.full((16,), dd, jnp.int32)
                        a = plsc.load_gather(srows, [row, col])
                        b = plsc.load_gather(drows, [row, col])
                        acc = acc + a * b
                    outv[pl.ds(pl.multiple_of(g * 16, 16), 16)] = acc
                    return carry2

                lax.fori_loop(0, _E // 16, group_body, 0)
                pltpu.sync_copy(outv, out_hbm.at[pl.ds(base, _E)])

            return carry

        lax.fori_loop(0, k_max, chunk_body, 0)

    return edge_dot


def kernel(x, edge_index):
    ei = edge_index.astype(jnp.int32)
    fn = _build(ei.shape[1], x.shape[0], x.shape[1])
    return fn(x, ei[0], ei[1])


# speedup vs baseline: 1.3252x; 1.1750x over previous
import functools

import jax
import jax.numpy as jnp
from jax import lax
from jax.experimental import pallas as pl
from jax.experimental.pallas import tpu as pltpu
from jax.experimental.pallas import tpu_sc as plsc

_E = 128
_G = 16


@functools.lru_cache(maxsize=None)
def _build(n_edges, n_nodes, d):
    info = plsc.get_sparse_core_info()
    ncores, nsub = info.num_cores, info.num_subcores
    nw = ncores * nsub
    n_chunks = n_edges // _E
    main_chunks = n_chunks // nw
    span = main_chunks * _E
    tail_chunks = n_chunks - main_chunks * nw
    tail_base = span * nw
    buf_e = span + _E
    mesh = plsc.VectorSubcoreMesh(core_axis_name="c", subcore_axis_name="s")

    @functools.partial(
        pl.kernel,
        out_type=jax.ShapeDtypeStruct((n_edges,), jnp.float32),
        mesh=mesh,
        compiler_params=pltpu.CompilerParams(needs_layout_passes=False),
        scratch_types=[
            pltpu.VMEM((buf_e,), jnp.int32),
            pltpu.VMEM((buf_e,), jnp.int32),
            pltpu.VMEM((_E, d), jnp.float32),
            pltpu.VMEM((_E, d), jnp.float32),
            pltpu.VMEM((_E, d), jnp.float32),
            pltpu.VMEM((_E, d), jnp.float32),
            pltpu.VMEM((buf_e,), jnp.float32),
            pltpu.SemaphoreType.DMA,
            pltpu.SemaphoreType.DMA,
        ],
    )
    def edge_dot(x_hbm, src_hbm, dst_hbm, out_hbm,
                 sidx, didx, sr0, dr0, sr1, dr1, outv, sem0, sem1):
        wid = lax.axis_index("s") * ncores + lax.axis_index("c")
        base = wid * span
        lanes = lax.iota(jnp.int32, _G)
        bufs = ((sr0, dr0, sem0), (sr1, dr1, sem1))

        pltpu.sync_copy(src_hbm.at[pl.ds(base, span)], sidx.at[pl.ds(0, span)])
        pltpu.sync_copy(dst_hbm.at[pl.ds(base, span)], didx.at[pl.ds(0, span)])

        def fire(c, b):
            sr, dr, sem = bufs[b]
            pltpu.async_copy(x_hbm.at[sidx.at[pl.ds(c * _E, _E)]], sr, sem)
            pltpu.async_copy(x_hbm.at[didx.at[pl.ds(c * _E, _E)]], dr, sem)

        def drain(b):
            sr, dr, sem = bufs[b]
            pltpu.make_async_copy(x_hbm.at[sidx.at[pl.ds(0, _E)]], sr, sem).wait()
            pltpu.make_async_copy(x_hbm.at[didx.at[pl.ds(0, _E)]], dr, sem).wait()

        def compute(c, b):
            sr, dr, _ = bufs[b]

            def group_body(g, carry):
                row = g * _G + lanes
                acc = jnp.zeros((_G,), jnp.float32)
                for dd in range(d):
                    col = jnp.full((_G,), dd, jnp.int32)
                    acc = acc + (plsc.load_gather(sr, [row, col])
                                 * plsc.load_gather(dr, [row, col]))
                outv[pl.ds(pl.multiple_of(c * _E, _E) + g * _G, _G)] = acc
                return carry

            lax.fori_loop(0, _E // _G, group_body, 0)

        fire(0, 0)

        def loop_body(i, carry):
            c0 = i * 2
            fire(c0 + 1, 1)
            drain(0)
            compute(c0, 0)

            @pl.when(c0 + 2 < main_chunks)
            def _():
                fire(c0 + 2, 0)

            drain(1)
            compute(c0 + 1, 1)
            return carry

        lax.fori_loop(0, main_chunks // 2, loop_body, 0)

        @pl.when(wid < tail_chunks)
        def _():
            tb = tail_base + wid * _E
            pltpu.sync_copy(src_hbm.at[pl.ds(tb, _E)], sidx.at[pl.ds(span, _E)])
            pltpu.sync_copy(dst_hbm.at[pl.ds(tb, _E)], didx.at[pl.ds(span, _E)])
            fire(main_chunks, 0)
            drain(0)
            compute(main_chunks, 0)
            pltpu.sync_copy(outv.at[pl.ds(span, _E)], out_hbm.at[pl.ds(tb, _E)])

        pltpu.sync_copy(outv.at[pl.ds(0, span)], out_hbm.at[pl.ds(base, span)])

    return edge_dot


def kernel(x, edge_index):
    ei = edge_index.astype(jnp.int32)
    fn = _build(ei.shape[1], x.shape[0], x.shape[1])
    return fn(x, ei[0], ei[1])


# speedup vs baseline: 4.9656x; 3.7470x over previous
import functools

import jax
import jax.numpy as jnp
from jax import lax
from jax.experimental import pallas as pl
from jax.experimental.pallas import tpu as pltpu
from jax.experimental.pallas import tpu_sc as plsc

_E = 128
_G = 16


@functools.lru_cache(maxsize=None)
def _build(n_edges, n_nodes, d):
    info = plsc.get_sparse_core_info()
    ncores, nsub = info.num_cores, info.num_subcores
    nw = ncores * nsub
    n_chunks = n_edges // _E
    main_chunks = n_chunks // nw
    span = main_chunks * _E
    tail_chunks = n_chunks - main_chunks * nw
    tail_base = span * nw
    buf_e = span + _E
    mesh = plsc.VectorSubcoreMesh(core_axis_name="c", subcore_axis_name="s")

    @functools.partial(
        pl.kernel,
        out_type=jax.ShapeDtypeStruct((n_edges,), jnp.float32),
        mesh=mesh,
        compiler_params=pltpu.CompilerParams(needs_layout_passes=False),
        scratch_types=[
            pltpu.VMEM((buf_e,), jnp.int32),
            pltpu.VMEM((buf_e,), jnp.int32),
            pltpu.VMEM((_E, d), jnp.float32),
            pltpu.VMEM((_E, d), jnp.float32),
            pltpu.VMEM((_E, d), jnp.float32),
            pltpu.VMEM((_E, d), jnp.float32),
            pltpu.VMEM((buf_e,), jnp.float32),
            pltpu.SemaphoreType.DMA,
            pltpu.SemaphoreType.DMA,
        ],
    )
    def edge_dot(x_hbm, src_hbm, dst_hbm, out_hbm,
                 sidx, didx, sr0, dr0, sr1, dr1, outv, sem0, sem1):
        wid = lax.axis_index("s") * ncores + lax.axis_index("c")
        base = wid * span
        lanes = lax.iota(jnp.int32, _G)
        bufs = ((sr0, dr0, sem0), (sr1, dr1, sem1))

        pltpu.sync_copy(src_hbm.at[pl.ds(base, span)], sidx.at[pl.ds(0, span)])
        pltpu.sync_copy(dst_hbm.at[pl.ds(base, span)], didx.at[pl.ds(0, span)])

        def fire(c, b):
            sr, dr, sem = bufs[b]
            pltpu.async_copy(x_hbm.at[sidx.at[pl.ds(c * _E, _E)]], sr, sem)
            pltpu.async_copy(x_hbm.at[didx.at[pl.ds(c * _E, _E)]], dr, sem)

        def drain(b):
            sr, dr, sem = bufs[b]
            pltpu.make_async_copy(x_hbm.at[sidx.at[pl.ds(0, _E)]], sr, sem).wait()
            pltpu.make_async_copy(x_hbm.at[didx.at[pl.ds(0, _E)]], dr, sem).wait()

        def compute(c, b):
            sr, dr, _ = bufs[b]

            def group_body(g, carry):
                row = g * _G + lanes
                acc = jnp.zeros((_G,), jnp.float32)
                for dd in range(d):
                    col = (lanes + dd) & (d - 1)
                    acc = acc + (plsc.load_gather(sr, [row, col])
                                 * plsc.load_gather(dr, [row, col]))
                outv[pl.ds(pl.multiple_of(c * _E, _E) + g * _G, _G)] = acc
                return carry

            lax.fori_loop(0, _E // _G, group_body, 0)

        fire(0, 0)

        def loop_body(i, carry):
            c0 = i * 2
            fire(c0 + 1, 1)
            drain(0)
            compute(c0, 0)

            @pl.when(c0 + 2 < main_chunks)
            def _():
                fire(c0 + 2, 0)

            drain(1)
            compute(c0 + 1, 1)
            return carry

        lax.fori_loop(0, main_chunks // 2, loop_body, 0)

        @pl.when(wid < tail_chunks)
        def _():
            tb = tail_base + wid * _E
            pltpu.sync_copy(src_hbm.at[pl.ds(tb, _E)], sidx.at[pl.ds(span, _E)])
            pltpu.sync_copy(dst_hbm.at[pl.ds(tb, _E)], didx.at[pl.ds(span, _E)])
            fire(main_chunks, 0)
            drain(0)
            compute(main_chunks, 0)
            pltpu.sync_copy(outv.at[pl.ds(span, _E)], out_hbm.at[pl.ds(tb, _E)])

        pltpu.sync_copy(outv.at[pl.ds(0, span)], out_hbm.at[pl.ds(base, span)])

    return edge_dot


def kernel(x, edge_index):
    ei = edge_index.astype(jnp.int32)
    fn = _build(ei.shape[1], x.shape[0], x.shape[1])
    return fn(x, ei[0], ei[1])
